# baseline (speedup 1.0000x reference)
import functools

import jax
import jax.numpy as jnp
from jax import lax
from jax.experimental import pallas as pl
from jax.experimental.pallas import tpu as pltpu
from jax.experimental.pallas import tpu_sc as plsc

VOCAB = 1000000
EMBED = 64
BATCH = 4096
SEQ = 200

GLEN = 104
REAL = 100
GROUPS_PER_ROW = 2
NC, NS = 2, 16
NW = NC * NS
ROWS_PER_W = BATCH // NW
GROUPS_PER_W = ROWS_PER_W * GROUPS_PER_ROW


def _start_gather(table_hbm, idx_v, buf, sem, r_local):
    g = r_local * GROUPS_PER_ROW
    pltpu.async_copy(table_hbm.at[idx_v.at[g]], buf.at[pl.ds(0, GLEN)], sem)
    pltpu.async_copy(table_hbm.at[idx_v.at[g + 1]], buf.at[pl.ds(GLEN, GLEN)], sem)


def _wait_gather(table_hbm, idx_v, buf, sem):
    pltpu.make_async_copy(
        table_hbm.at[idx_v.at[0]], buf.at[pl.ds(0, GLEN)], sem).wait()
    pltpu.make_async_copy(
        table_hbm.at[idx_v.at[0]], buf.at[pl.ds(GLEN, GLEN)], sem).wait()


def _reduce_buf(buf, out_v, r_local):
    zero = jnp.zeros((16,), jnp.float32)

    def body(j, accs):
        new = []
        for c in range(EMBED // 16):
            sl = pl.ds(c * 16, 16)
            new.append(accs[c] + buf[j, sl] + buf[GLEN + j, sl])
        return tuple(new)

    accs = lax.fori_loop(0, REAL, body, (zero,) * (EMBED // 16))
    for c in range(EMBED // 16):
        out_v[r_local, pl.ds(c * 16, 16)] = accs[c]


@functools.partial(
    pl.kernel,
    mesh=plsc.VectorSubcoreMesh(core_axis_name="c", subcore_axis_name="s"),
    out_type=jax.ShapeDtypeStruct((BATCH, EMBED), jnp.float32),
    scratch_types=[
        pltpu.VMEM((GROUPS_PER_W, GLEN), jnp.int32),
        pltpu.VMEM((2 * GLEN, EMBED), jnp.float32),
        pltpu.VMEM((2 * GLEN, EMBED), jnp.float32),
        pltpu.VMEM((ROWS_PER_W, EMBED), jnp.float32),
        pltpu.SemaphoreType.DMA,
        pltpu.SemaphoreType.DMA,
    ],
    compiler_params=pltpu.CompilerParams(use_tc_tiling_on_sc=False),
)
def _sc_pool(idx_hbm, table_hbm, out_hbm, idx_v, buf_a, buf_b, out_v,
             sem_a, sem_b):
    wid = lax.axis_index("s") * NC + lax.axis_index("c")
    base_g = wid * GROUPS_PER_W
    base_r = wid * ROWS_PER_W

    pltpu.sync_copy(idx_hbm.at[pl.ds(base_g, GROUPS_PER_W)], idx_v)

    _start_gather(table_hbm, idx_v, buf_a, sem_a, 0)

    def step(i, carry):
        r0 = 2 * i
        _start_gather(table_hbm, idx_v, buf_b, sem_b, r0 + 1)
        _wait_gather(table_hbm, idx_v, buf_a, sem_a)
        _reduce_buf(buf_a, out_v, r0)
        r_next = jnp.minimum(r0 + 2, ROWS_PER_W - 2)
        _start_gather(table_hbm, idx_v, buf_a, sem_a, r_next)
        _wait_gather(table_hbm, idx_v, buf_b, sem_b)
        _reduce_buf(buf_b, out_v, r0 + 1)
        return carry

    lax.fori_loop(0, ROWS_PER_W // 2, step, 0)

    _wait_gather(table_hbm, idx_v, buf_a, sem_a)

    pltpu.sync_copy(out_v, out_hbm.at[pl.ds(base_r, ROWS_PER_W)])


def _mlp_body(ps_ref, w1_ref, b1_ref, w2_ref, b2_ref, o_ref):
    pooled = ps_ref[...] * (1.0 / SEQ)
    h = jnp.dot(pooled, w1_ref[...], preferred_element_type=jnp.float32)
    h = jnp.maximum(h + b1_ref[...], 0.0)
    z = jnp.sum(h * w2_ref[...], axis=1, keepdims=True) + b2_ref[...]
    o_ref[...] = 1.0 / (1.0 + jnp.exp(-z))


def _mlp(pooled_sum, W1, b1, W2, b2):
    return pl.pallas_call(
        _mlp_body,
        out_shape=jax.ShapeDtypeStruct((BATCH, 1), jnp.float32),
    )(pooled_sum, W1, b1.reshape(1, EMBED), W2.reshape(1, EMBED),
      b2.reshape(1, 1))


def kernel(x, table, W1, b1, W2, b2):
    xg = x.astype(jnp.int32).reshape(BATCH, GROUPS_PER_ROW, REAL)
    xg = jnp.pad(xg, ((0, 0), (0, 0), (0, GLEN - REAL)))
    idx = xg.reshape(BATCH * GROUPS_PER_ROW, GLEN)
    pooled_sum = _sc_pool(idx, table)
    return _mlp(pooled_sum, W1, b1, W2, b2)
